# baseline (speedup 1.0000x reference)
import functools

import jax
import jax.numpy as jnp
from jax import lax
from jax.experimental import pallas as pl
from jax.experimental.pallas import tpu as pltpu
from jax.experimental.pallas import tpu_sc as plsc

_NS = 16
_L = 16
_B = 512
_BH = _B // 4

_C0 = (0, 0, 0, 0, 0, 0, 0, 0, 1, 1, 1, 1, 1, 1, 1, 1)
_C1 = (0, 0, 1, 1, 0, 0, 1, 1, -1, -1, 0, 0, -1, -1, 0, 0)
_C2 = (0, 0, 0, 0, 1, 1, 1, 1, -1, -1, -1, -1, 0, 0, 0, 0)
_C3 = (0, 1, -1, 0, -1, 0, -2, -1, 1, 2, 0, 1, 0, 1, -1, 0)


@functools.lru_cache(maxsize=None)
def _make_layer(din, dout, k_chunk):
    n_w = dout // _NS
    n_chunks = n_w // k_chunk
    mesh = plsc.VectorSubcoreMesh(core_axis_name="c", subcore_axis_name="s")

    @functools.partial(
        pl.kernel, mesh=mesh,
        out_type=jax.ShapeDtypeStruct((2, dout, _BH), jnp.int32),
        compiler_params=pltpu.CompilerParams(needs_layout_passes=False),
        scratch_types=[
            pltpu.VMEM((n_w * 16,), jnp.float32),
            pltpu.VMEM((n_w,), jnp.float32),
            pltpu.VMEM((n_w,), jnp.float32),
            pltpu.VMEM((n_w,), jnp.float32),
            pltpu.VMEM((n_w,), jnp.float32),
            pltpu.VMEM((n_w,), jnp.int32),
            pltpu.VMEM((n_w,), jnp.int32),
            pltpu.VMEM((k_chunk, _BH), jnp.int32),
            pltpu.VMEM((k_chunk, _BH), jnp.int32),
            pltpu.VMEM((k_chunk, _BH), jnp.int32),
            pltpu.VMEM((k_chunk, _BH), jnp.int32),
            pltpu.VMEM((k_chunk, _BH), jnp.int32),
            pltpu.VMEM((k_chunk, _BH), jnp.int32),
            pltpu.SemaphoreType.DMA,
            pltpu.SemaphoreType.DMA,
            pltpu.SemaphoreType.DMA,
            pltpu.SemaphoreType.DMA,
            pltpu.SemaphoreType.DMA,
            pltpu.SemaphoreType.DMA,
        ],
    )
    def layer(xt, w, ia, ib, out, wv, t0, t1, t2, t3, iav, ibv,
              av0, av1, bv0, bv1, ov0, ov1, sa0, sa1, sb0, sb1, so0, so1):
        c = lax.axis_index("c")
        t = lax.axis_index("s")
        base = t * n_w
        src = xt.at[c]
        dst = out.at[c]
        pltpu.sync_copy(ia.at[pl.ds(base, n_w)], iav)
        pltpu.sync_copy(ib.at[pl.ds(base, n_w)], ibv)
        pltpu.sync_copy(w.at[pl.ds(base * 16, n_w * 16)], wv)

        abufs, bbufs, obufs = (av0, av1), (bv0, bv1), (ov0, ov1)
        asems, bsems, osems = (sa0, sa1), (sb0, sb1), (so0, so1)

        def issue_gather(ck):
            p = ck % 2
            sl = pl.ds(ck * k_chunk, k_chunk)
            ha = pltpu.async_copy(src.at[iav.at[sl]], abufs[p], asems[p])
            hb = pltpu.async_copy(src.at[ibv.at[sl]], bbufs[p], bsems[p])
            return ha, hb

        pend = {0: issue_gather(0)}
        if n_chunks > 1:
            pend[1] = issue_gather(1)

        lane = jnp.arange(_L, dtype=jnp.int32)

        def coef_body(g, carry):
            idxr = (g * _L + lane) * 16
            rows = [plsc.load_gather(wv, [idxr + i]) for i in range(16)]
            m = rows[0]
            for r in rows[1:]:
                m = jnp.maximum(m, r)
            es = [jnp.exp(r - m) for r in rows]
            s = es[0]
            for e in es[1:]:
                s = s + e
            inv = 1.0 / s

            def mix(coefs):
                acc = None
                for cf, e in zip(coefs, es):
                    if cf == 0:
                        continue
                    term = e if cf == 1 else (-e if cf == -1 else cf * e)
                    acc = term if acc is None else acc + term
                return acc * inv

            sl = pl.ds(g * _L, _L)
            t0[sl] = mix(_C0)
            t1[sl] = mix(_C1)
            t2[sl] = mix(_C2)
            t3[sl] = mix(_C3)
            return carry

        lax.fori_loop(0, n_w // _L, coef_body, 0)

        fmt = plsc.PackFormat.INTERLEAVED
        owaits = {}
        for ck in range(n_chunks):
            p = ck % 2
            ha, hb = pend.pop(ck)
            ha.wait()
            hb.wait()
            if ck - 2 in owaits:
                owaits.pop(ck - 2).wait()
            av, bv, ov = abufs[p], bbufs[p], obufs[p]

            def neuron_body(j, carry2, _ck=ck, _av=av, _bv=bv, _ov=ov):
                jj = _ck * k_chunk + j
                idx = jnp.full((_L,), jj, dtype=jnp.int32)
                c0f = plsc.load_gather(t0, [idx])
                c1f = plsc.load_gather(t1, [idx])
                c2f = plsc.load_gather(t2, [idx])
                c3f = plsc.load_gather(t3, [idx])
                c0 = plsc.pack(c0f, c0f, format=fmt)
                c1 = plsc.pack(c1f, c1f, format=fmt)
                c2 = plsc.pack(c2f, c2f, format=fmt)
                c3 = plsc.pack(c3f, c3f, format=fmt)
                for v in range(_BH // _L):
                    sl = pl.ds(v * _L, _L)
                    a = plsc.bitcast(_av[j, sl], jnp.bfloat16)
                    b = plsc.bitcast(_bv[j, sl], jnp.bfloat16)
                    r = (c0 + c1 * a) + (c2 + c3 * a) * b
                    _ov[j, sl] = plsc.bitcast(r, jnp.int32)
                return carry2

            lax.fori_loop(0, k_chunk, neuron_body, 0)
            owaits[ck] = pltpu.async_copy(
                ov, dst.at[pl.ds(base + ck * k_chunk, k_chunk)], osems[p])
            if ck + 2 < n_chunks:
                pend[ck + 2] = issue_gather(ck + 2)
        for h in owaits.values():
            h.wait()

    return layer


_DIMS = ((1024, 8192), (8192, 8192), (8192, 8192), (8192, 512))


def kernel(x, w0, a0, b0, w1, a1, b1, w2, a2, b2, w3, a3, b3):
    ws = (w0, w1, w2, w3)
    ias = (a0, a1, a2, a3)
    ibs = (b0, b1, b2, b3)
    xb = x.T.astype(jnp.bfloat16).reshape(_DIMS[0][0], 2, _BH, 2)
    h = lax.bitcast_convert_type(xb, jnp.int32).transpose(1, 0, 2)
    for i, (din, dout) in enumerate(_DIMS):
        n_w = dout // _NS
        k_chunk = min(64, n_w)
        layer = _make_layer(din, dout, k_chunk)
        h = layer(h, ws[i].reshape(dout * 16), ias[i], ibs[i])
    ob = lax.bitcast_convert_type(h, jnp.bfloat16)
    out = ob.reshape(2, _DIMS[-1][1], _B // 2).transpose(0, 2, 1)
    return out.reshape(_B, _DIMS[-1][1]).astype(jnp.float32)


# speedup vs baseline: 1.8746x; 1.8746x over previous
import functools

import jax
import jax.numpy as jnp
from jax import lax
from jax.experimental import pallas as pl
from jax.experimental.pallas import tpu as pltpu
from jax.experimental.pallas import tpu_sc as plsc

_NC = 2
_NS = 16
_NW = _NC * _NS
_L = 16
_B = 512
_B2 = _B // 2

_C0 = (0, 0, 0, 0, 0, 0, 0, 0, 1, 1, 1, 1, 1, 1, 1, 1)
_C1 = (0, 0, 1, 1, 0, 0, 1, 1, -1, -1, 0, 0, -1, -1, 0, 0)
_C2 = (0, 0, 0, 0, 1, 1, 1, 1, -1, -1, -1, -1, 0, 0, 0, 0)
_C3 = (0, 1, -1, 0, -1, 0, -2, -1, 1, 2, 0, 1, 0, 1, -1, 0)

_DIMS = ((1024, 8192), (8192, 8192), (8192, 8192), (8192, 512))
_K = 64


def _coef_prep(wv, ts, n_w):
    lane = jnp.arange(_L, dtype=jnp.int32)

    def coef_body(g, carry):
        idxr = (g * _L + lane) * 16
        rows = [plsc.load_gather(wv, [idxr + i]) for i in range(16)]
        m = rows[0]
        for r in rows[1:]:
            m = jnp.maximum(m, r)
        es = [jnp.exp(r - m) for r in rows]
        s = es[0]
        for e in es[1:]:
            s = s + e
        inv = 1.0 / s

        def mix(coefs):
            acc = None
            for cf, e in zip(coefs, es):
                if cf == 0:
                    continue
                term = e if cf == 1 else (-e if cf == -1 else cf * e)
                acc = term if acc is None else acc + term
            return acc * inv

        sl = pl.ds(g * _L, _L)
        ts[0][sl] = mix(_C0)
        ts[1][sl] = mix(_C1)
        ts[2][sl] = mix(_C2)
        ts[3][sl] = mix(_C3)
        return carry

    lax.fori_loop(0, n_w // _L, coef_body, 0)


def _build():
    mesh = plsc.VectorSubcoreMesh(core_axis_name="c", subcore_axis_name="s")

    scratch = []
    for din, dout in _DIMS:
        n_w = dout // _NW
        scratch.append(pltpu.VMEM((n_w * 16,), jnp.float32))
        scratch += [pltpu.VMEM((n_w,), jnp.float32)] * 4
        scratch += [pltpu.VMEM((n_w,), jnp.int32)] * 2
    scratch += [pltpu.VMEM((_K, _B2), jnp.int32)] * 6
    scratch += [pltpu.VMEM((_L,), jnp.int32)] * 2
    scratch += [pltpu.SemaphoreType.DMA] * 6

    @functools.partial(
        pl.kernel, mesh=mesh,
        out_type=(
            jax.ShapeDtypeStruct((_DIMS[0][1], _B2), jnp.int32),
            jax.ShapeDtypeStruct((_DIMS[1][1], _B2), jnp.int32),
            jax.ShapeDtypeStruct((_DIMS[3][1], _B2), jnp.int32),
            jax.ShapeDtypeStruct((128,), jnp.int32),
        ),
        compiler_params=pltpu.CompilerParams(needs_layout_passes=False),
        scratch_types=scratch,
    )
    def fused(xt, w0, ia0, ib0, w1, ia1, ib1, w2, ia2, ib2, w3, ia3, ib3,
              h1, h2, hout, flags, *sc):
        per_layer, rest = sc[:28], sc[28:]
        wvs = [per_layer[i * 7] for i in range(4)]
        tss = [per_layer[i * 7 + 1:i * 7 + 5] for i in range(4)]
        iavs = [per_layer[i * 7 + 5] for i in range(4)]
        ibvs = [per_layer[i * 7 + 6] for i in range(4)]
        av0, av1, bv0, bv1, ov0, ov1, flagv, pollv = rest[:8]
        sa0, sa1, sb0, sb1, so0, so1 = rest[8:14]
        abufs, bbufs, obufs = (av0, av1), (bv0, bv1), (ov0, ov1)
        asems, bsems, osems = (sa0, sa1), (sb0, sb1), (so0, so1)

        c = lax.axis_index("c")
        s = lax.axis_index("s")
        wid = s * _NC + c
        ws = (w0, w1, w2, w3)
        ias = (ia0, ia1, ia2, ia3)
        ibs = (ib0, ib1, ib2, ib3)
        srcs = (xt, h1, h2, h1)
        dsts = (h1, h2, h1, hout)

        @pl.when(s == 0)
        def _():
            flagv[...] = jnp.zeros((_L,), jnp.int32)
            for li in range(3):
                pltpu.sync_copy(
                    flagv, flags.at[pl.ds((li * _NC + c) * _L, _L)])

        def issue_gather(li, ck, k):
            src = srcs[li]
            p = ck % 2
            sl = pl.ds(ck * k, k)
            ha = pltpu.async_copy(
                src.at[iavs[li].at[sl]], abufs[p].at[pl.ds(0, k)], asems[p])
            hb = pltpu.async_copy(
                src.at[ibvs[li].at[sl]], bbufs[p].at[pl.ds(0, k)], bsems[p])
            return ha, hb

        for li, (din, dout) in enumerate(_DIMS):
            n_w = dout // _NW
            base = wid * n_w
            pltpu.sync_copy(ias[li].at[pl.ds(base, n_w)], iavs[li])
            pltpu.sync_copy(ibs[li].at[pl.ds(base, n_w)], ibvs[li])
            pltpu.sync_copy(ws[li].at[pl.ds(base * 16, n_w * 16)], wvs[li])

        first_k = min(_K, _DIMS[0][1] // _NW)
        pend = {(0, 0): issue_gather(0, 0, first_k),
                (0, 1): issue_gather(0, 1, first_k)}

        for li, (din, dout) in enumerate(_DIMS):
            _coef_prep(wvs[li], tss[li], dout // _NW)

        fmt = plsc.PackFormat.INTERLEAVED
        owaits = {}
        for li, (din, dout) in enumerate(_DIMS):
            n_w = dout // _NW
            k = min(_K, n_w)
            n_chunks = n_w // k
            base = wid * n_w
            t0, t1, t2, t3 = tss[li]
            dst = dsts[li]
            for ck in range(n_chunks):
                p = ck % 2
                ha, hb = pend.pop((li, ck))
                ha.wait()
                hb.wait()
                if (li, ck - 2) in owaits:
                    owaits.pop((li, ck - 2)).wait()
                av, bv, ov = abufs[p], bbufs[p], obufs[p]

                def neuron_body(j, carry, _ck=ck, _k=k, _av=av, _bv=bv,
                                _ov=ov, _t0=t0, _t1=t1, _t2=t2, _t3=t3):
                    jj = _ck * _k + j
                    idx = jnp.full((_L,), jj, dtype=jnp.int32)
                    c0f = plsc.load_gather(_t0, [idx])
                    c1f = plsc.load_gather(_t1, [idx])
                    c2f = plsc.load_gather(_t2, [idx])
                    c3f = plsc.load_gather(_t3, [idx])
                    c0 = plsc.pack(c0f, c0f, format=fmt)
                    c1 = plsc.pack(c1f, c1f, format=fmt)
                    c2 = plsc.pack(c2f, c2f, format=fmt)
                    c3 = plsc.pack(c3f, c3f, format=fmt)
                    for v in range(_B2 // _L):
                        sl = pl.ds(v * _L, _L)
                        a = plsc.bitcast(_av[j, sl], jnp.bfloat16)
                        b = plsc.bitcast(_bv[j, sl], jnp.bfloat16)
                        r = (c0 + c1 * a) + (c2 + c3 * a) * b
                        _ov[j, sl] = plsc.bitcast(r, jnp.int32)
                    return carry

                lax.fori_loop(0, k, neuron_body, 0)
                owaits[(li, ck)] = pltpu.async_copy(
                    ov.at[pl.ds(0, k)], dst.at[pl.ds(base + ck * k, k)],
                    osems[p])
                if ck + 2 < n_chunks:
                    pend[(li, ck + 2)] = issue_gather(li, ck + 2, k)
            for key in list(owaits):
                owaits.pop(key).wait()
            if li + 1 < len(_DIMS):
                plsc.subcore_barrier()

                @pl.when(s == 0)
                def _(_li=li):
                    flagv[...] = jnp.full((_L,), _li + 1, jnp.int32)
                    pltpu.sync_copy(
                        flagv, flags.at[pl.ds((_li * _NC + c) * _L, _L)])
                    other = pl.ds((_li * _NC + (1 - c)) * _L, _L)

                    def poll(val):
                        pltpu.sync_copy(flags.at[other], pollv)
                        return jnp.max(pollv[...], axis=0)

                    lax.while_loop(lambda v: v <= _li, poll,
                                   jnp.zeros((), jnp.int32))

                plsc.subcore_barrier()
                n_w2 = _DIMS[li + 1][1] // _NW
                k2 = min(_K, n_w2)
                pend[(li + 1, 0)] = issue_gather(li + 1, 0, k2)
                if n_w2 // k2 > 1:
                    pend[(li + 1, 1)] = issue_gather(li + 1, 1, k2)

    return fused


_FUSED = _build()


def kernel(x, w0, a0, b0, w1, a1, b1, w2, a2, b2, w3, a3, b3):
    xb = x.T.astype(jnp.bfloat16).reshape(_DIMS[0][0], _B2, 2)
    xt = lax.bitcast_convert_type(xb, jnp.int32)
    _, _, ho, _ = _FUSED(
        xt, w0.reshape(-1), a0, b0, w1.reshape(-1), a1, b1,
        w2.reshape(-1), a2, b2, w3.reshape(-1), a3, b3)
    out = lax.bitcast_convert_type(ho, jnp.bfloat16).reshape(_DIMS[-1][1], _B)
    return out.T.astype(jnp.float32)


# speedup vs baseline: 1.8788x; 1.0022x over previous
import functools

import jax
import jax.numpy as jnp
from jax import lax
from jax.experimental import pallas as pl
from jax.experimental.pallas import tpu as pltpu
from jax.experimental.pallas import tpu_sc as plsc

_NC = 2
_NS = 16
_NW = _NC * _NS
_L = 16
_B = 512
_B2 = _B // 2

_C0 = (0, 0, 0, 0, 0, 0, 0, 0, 1, 1, 1, 1, 1, 1, 1, 1)
_C1 = (0, 0, 1, 1, 0, 0, 1, 1, -1, -1, 0, 0, -1, -1, 0, 0)
_C2 = (0, 0, 0, 0, 1, 1, 1, 1, -1, -1, -1, -1, 0, 0, 0, 0)
_C3 = (0, 1, -1, 0, -1, 0, -2, -1, 1, 2, 0, 1, 0, 1, -1, 0)

_DIMS = ((1024, 8192), (8192, 8192), (8192, 8192), (8192, 512))
_K = 64


def _coef_prep(wv, ts, n_w):
    lane = jnp.arange(_L, dtype=jnp.int32)

    def coef_body(g, carry):
        idxr = (g * _L + lane) * 16
        rows = [plsc.load_gather(wv, [idxr + i]) for i in range(16)]
        m = rows[0]
        for r in rows[1:]:
            m = jnp.maximum(m, r)
        es = [jnp.exp(r - m) for r in rows]
        s = es[0]
        for e in es[1:]:
            s = s + e
        inv = 1.0 / s

        def mix(coefs):
            acc = None
            for cf, e in zip(coefs, es):
                if cf == 0:
                    continue
                term = e if cf == 1 else (-e if cf == -1 else cf * e)
                acc = term if acc is None else acc + term
            return acc * inv

        sl = pl.ds(g * _L, _L)
        ts[0][sl] = mix(_C0)
        ts[1][sl] = mix(_C1)
        ts[2][sl] = mix(_C2)
        ts[3][sl] = mix(_C3)
        return carry

    lax.fori_loop(0, n_w // _L, coef_body, 0)


def _build():
    mesh = plsc.VectorSubcoreMesh(core_axis_name="c", subcore_axis_name="s")

    scratch = []
    for din, dout in _DIMS:
        n_w = dout // _NW
        scratch.append(pltpu.VMEM((n_w * 16,), jnp.float32))
        scratch += [pltpu.VMEM((n_w,), jnp.float32)] * 4
        scratch += [pltpu.VMEM((n_w,), jnp.int32)] * 2
    scratch += [pltpu.VMEM((_K, _B2), jnp.int32)] * 6
    scratch += [pltpu.VMEM((_L,), jnp.int32)] * 2
    scratch += [pltpu.SemaphoreType.DMA] * 6
    scratch += [pltpu.HBM((_DIMS[1][1], _B2), jnp.int32)] * 2

    @functools.partial(
        pl.kernel, mesh=mesh,
        out_type=(
            jax.ShapeDtypeStruct((_DIMS[3][1], _B2), jnp.int32),
            jax.ShapeDtypeStruct((128,), jnp.int32),
        ),
        compiler_params=pltpu.CompilerParams(needs_layout_passes=False),
        scratch_types=scratch,
    )
    def fused(xt, w0, ia0, ib0, w1, ia1, ib1, w2, ia2, ib2, w3, ia3, ib3,
              hout, flags, *sc):
        h1, h2 = sc[-2:]
        sc = sc[:-2]
        per_layer, rest = sc[:28], sc[28:]
        wvs = [per_layer[i * 7] for i in range(4)]
        tss = [per_layer[i * 7 + 1:i * 7 + 5] for i in range(4)]
        iavs = [per_layer[i * 7 + 5] for i in range(4)]
        ibvs = [per_layer[i * 7 + 6] for i in range(4)]
        av0, av1, bv0, bv1, ov0, ov1, flagv, pollv = rest[:8]
        sa0, sa1, sb0, sb1, so0, so1 = rest[8:14]
        abufs, bbufs, obufs = (av0, av1), (bv0, bv1), (ov0, ov1)
        asems, bsems, osems = (sa0, sa1), (sb0, sb1), (so0, so1)

        c = lax.axis_index("c")
        s = lax.axis_index("s")
        wid = s * _NC + c
        ws = (w0, w1, w2, w3)
        ias = (ia0, ia1, ia2, ia3)
        ibs = (ib0, ib1, ib2, ib3)
        srcs = (xt, h1, h2, h1)
        dsts = (h1, h2, h1, hout)

        @pl.when(s == 0)
        def _():
            flagv[...] = jnp.zeros((_L,), jnp.int32)
            for li in range(3):
                pltpu.sync_copy(
                    flagv, flags.at[pl.ds((li * _NC + c) * _L, _L)])

        def issue_gather(li, ck, k):
            src = srcs[li]
            p = ck % 2
            sl = pl.ds(ck * k, k)
            ha = pltpu.async_copy(
                src.at[iavs[li].at[sl]], abufs[p].at[pl.ds(0, k)], asems[p])
            hb = pltpu.async_copy(
                src.at[ibvs[li].at[sl]], bbufs[p].at[pl.ds(0, k)], bsems[p])
            return ha, hb

        for li, (din, dout) in enumerate(_DIMS):
            n_w = dout // _NW
            base = wid * n_w
            pltpu.sync_copy(ias[li].at[pl.ds(base, n_w)], iavs[li])
            pltpu.sync_copy(ibs[li].at[pl.ds(base, n_w)], ibvs[li])
            pltpu.sync_copy(ws[li].at[pl.ds(base * 16, n_w * 16)], wvs[li])

        first_k = min(_K, _DIMS[0][1] // _NW)
        pend = {(0, 0): issue_gather(0, 0, first_k),
                (0, 1): issue_gather(0, 1, first_k)}

        for li, (din, dout) in enumerate(_DIMS):
            _coef_prep(wvs[li], tss[li], dout // _NW)

        fmt = plsc.PackFormat.INTERLEAVED
        owaits = {}
        for li, (din, dout) in enumerate(_DIMS):
            n_w = dout // _NW
            k = min(_K, n_w)
            n_chunks = n_w // k
            base = wid * n_w
            t0, t1, t2, t3 = tss[li]
            dst = dsts[li]
            for ck in range(n_chunks):
                p = ck % 2
                ha, hb = pend.pop((li, ck))
                ha.wait()
                hb.wait()
                if (li, ck - 2) in owaits:
                    owaits.pop((li, ck - 2)).wait()
                av, bv, ov = abufs[p], bbufs[p], obufs[p]

                def neuron_body(j, carry, _ck=ck, _k=k, _av=av, _bv=bv,
                                _ov=ov, _t0=t0, _t1=t1, _t2=t2, _t3=t3):
                    jj = _ck * _k + j
                    idx = jnp.full((_L,), jj, dtype=jnp.int32)
                    c0f = plsc.load_gather(_t0, [idx])
                    c1f = plsc.load_gather(_t1, [idx])
                    c2f = plsc.load_gather(_t2, [idx])
                    c3f = plsc.load_gather(_t3, [idx])
                    c0 = plsc.pack(c0f, c0f, format=fmt)
                    c1 = plsc.pack(c1f, c1f, format=fmt)
                    c2 = plsc.pack(c2f, c2f, format=fmt)
                    c3 = plsc.pack(c3f, c3f, format=fmt)
                    for v in range(_B2 // _L):
                        sl = pl.ds(v * _L, _L)
                        a = plsc.bitcast(_av[j, sl], jnp.bfloat16)
                        b = plsc.bitcast(_bv[j, sl], jnp.bfloat16)
                        r = (c0 + c1 * a) + (c2 + c3 * a) * b
                        _ov[j, sl] = plsc.bitcast(r, jnp.int32)
                    return carry

                lax.fori_loop(0, k, neuron_body, 0)
                owaits[(li, ck)] = pltpu.async_copy(
                    ov.at[pl.ds(0, k)], dst.at[pl.ds(base + ck * k, k)],
                    osems[p])
                if ck + 2 < n_chunks:
                    pend[(li, ck + 2)] = issue_gather(li, ck + 2, k)
            for key in list(owaits):
                owaits.pop(key).wait()
            if li + 1 < len(_DIMS):
                plsc.subcore_barrier()

                @pl.when(s == 0)
                def _(_li=li):
                    flagv[...] = jnp.full((_L,), _li + 1, jnp.int32)
                    pltpu.sync_copy(
                        flagv, flags.at[pl.ds((_li * _NC + c) * _L, _L)])
                    other = pl.ds((_li * _NC + (1 - c)) * _L, _L)

                    def poll(val):
                        pltpu.sync_copy(flags.at[other], pollv)
                        return jnp.max(pollv[...], axis=0)

                    lax.while_loop(lambda v: v <= _li, poll,
                                   jnp.zeros((), jnp.int32))

                plsc.subcore_barrier()
                n_w2 = _DIMS[li + 1][1] // _NW
                k2 = min(_K, n_w2)
                pend[(li + 1, 0)] = issue_gather(li + 1, 0, k2)
                if n_w2 // k2 > 1:
                    pend[(li + 1, 1)] = issue_gather(li + 1, 1, k2)

    return fused


_FUSED = _build()


def kernel(x, w0, a0, b0, w1, a1, b1, w2, a2, b2, w3, a3, b3):
    xb = x.T.astype(jnp.bfloat16).reshape(_DIMS[0][0], _B2, 2)
    xt = lax.bitcast_convert_type(xb, jnp.int32)
    ho, _ = _FUSED(
        xt, w0.reshape(-1), a0, b0, w1.reshape(-1), a1, b1,
        w2.reshape(-1), a2, b2, w3.reshape(-1), a3, b3)
    out = lax.bitcast_convert_type(ho, jnp.bfloat16).reshape(_DIMS[-1][1], _B)
    return out.T.astype(jnp.float32)


# speedup vs baseline: 1.9258x; 1.0250x over previous
import functools

import jax
import jax.numpy as jnp
from jax import lax
from jax.experimental import pallas as pl
from jax.experimental.pallas import tpu as pltpu
from jax.experimental.pallas import tpu_sc as plsc

_NC = 2
_NS = 16
_NW = _NC * _NS
_L = 16
_B = 512
_B2 = _B // 2

_C0 = (0, 0, 0, 0, 0, 0, 0, 0, 1, 1, 1, 1, 1, 1, 1, 1)
_C1 = (0, 0, 1, 1, 0, 0, 1, 1, -1, -1, 0, 0, -1, -1, 0, 0)
_C2 = (0, 0, 0, 0, 1, 1, 1, 1, -1, -1, -1, -1, 0, 0, 0, 0)
_C3 = (0, 1, -1, 0, -1, 0, -2, -1, 1, 2, 0, 1, 0, 1, -1, 0)

_DIMS = ((1024, 8192), (8192, 8192), (8192, 8192), (8192, 512))
_K = 64


def _coef_prep(wv, ts, n_w):
    lane = jnp.arange(_L, dtype=jnp.int32)

    def coef_body(g, carry):
        idxr = (g * _L + lane) * 16
        rows = [plsc.load_gather(wv, [idxr + i]) for i in range(16)]
        m = rows[0]
        for r in rows[1:]:
            m = jnp.maximum(m, r)
        es = [jnp.exp(r - m) for r in rows]
        s = es[0]
        for e in es[1:]:
            s = s + e
        inv = 1.0 / s

        def mix(coefs):
            acc = None
            for cf, e in zip(coefs, es):
                if cf == 0:
                    continue
                term = e if cf == 1 else (-e if cf == -1 else cf * e)
                acc = term if acc is None else acc + term
            return acc * inv

        sl = pl.ds(g * _L, _L)
        ts[0][sl] = mix(_C0)
        ts[1][sl] = mix(_C1)
        ts[2][sl] = mix(_C2)
        ts[3][sl] = mix(_C3)
        return carry

    lax.fori_loop(0, n_w // _L, coef_body, 0)


def _build():
    mesh = plsc.VectorSubcoreMesh(core_axis_name="c", subcore_axis_name="s")

    scratch = []
    for din, dout in _DIMS:
        n_w = dout // _NW
        scratch.append(pltpu.VMEM((n_w * 16,), jnp.float32))
        scratch += [pltpu.VMEM((n_w,), jnp.float32)] * 4
        scratch += [pltpu.VMEM((n_w,), jnp.int32)] * 2
    scratch += [pltpu.VMEM((_K, _B2), jnp.int32)] * 6
    scratch += [pltpu.VMEM((_L,), jnp.int32)] * 2
    scratch += [pltpu.SemaphoreType.DMA] * 8
    scratch += [pltpu.HBM((_DIMS[1][1], _B2), jnp.int32)] * 2

    @functools.partial(
        pl.kernel, mesh=mesh,
        out_type=(
            jax.ShapeDtypeStruct((_DIMS[3][1], _B2), jnp.int32),
            jax.ShapeDtypeStruct((128,), jnp.int32),
        ),
        compiler_params=pltpu.CompilerParams(needs_layout_passes=False),
        scratch_types=scratch,
    )
    def fused(xt, w0, ia0, ib0, w1, ia1, ib1, w2, ia2, ib2, w3, ia3, ib3,
              hout, flags, *sc):
        h1, h2 = sc[-2:]
        sc = sc[:-2]
        per_layer, rest = sc[:28], sc[28:]
        wvs = [per_layer[i * 7] for i in range(4)]
        tss = [per_layer[i * 7 + 1:i * 7 + 5] for i in range(4)]
        iavs = [per_layer[i * 7 + 5] for i in range(4)]
        ibvs = [per_layer[i * 7 + 6] for i in range(4)]
        av0, av1, bv0, bv1, ov0, ov1, flagv, pollv = rest[:8]
        sa0, sa1, sb0, sb1, so0, so1, sidx, sw = rest[8:16]
        abufs, bbufs, obufs = (av0, av1), (bv0, bv1), (ov0, ov1)
        asems, bsems, osems = (sa0, sa1), (sb0, sb1), (so0, so1)

        c = lax.axis_index("c")
        s = lax.axis_index("s")
        wid = s * _NC + c
        ws = (w0, w1, w2, w3)
        ias = (ia0, ia1, ia2, ia3)
        ibs = (ib0, ib1, ib2, ib3)
        srcs = (xt, h1, h2, h1)
        dsts = (h1, h2, h1, hout)

        @pl.when(s == 0)
        def _():
            flagv[...] = jnp.zeros((_L,), jnp.int32)
            for li in range(3):
                pltpu.sync_copy(
                    flagv, flags.at[pl.ds((li * _NC + c) * _L, _L)])

        def issue_gather(li, ck, k):
            src = srcs[li]
            p = ck % 2
            sl = pl.ds(ck * k, k)
            ha = pltpu.async_copy(
                src.at[iavs[li].at[sl]], abufs[p].at[pl.ds(0, k)], asems[p])
            hb = pltpu.async_copy(
                src.at[ibvs[li].at[sl]], bbufs[p].at[pl.ds(0, k)], bsems[p])
            return ha, hb

        stage = []
        for li, (din, dout) in enumerate(_DIMS):
            n_w = dout // _NW
            base = wid * n_w
            stage.append(pltpu.async_copy(
                ias[li].at[pl.ds(base, n_w)], iavs[li], sidx))
            stage.append(pltpu.async_copy(
                ibs[li].at[pl.ds(base, n_w)], ibvs[li], sidx))
            stage.append(pltpu.async_copy(
                ws[li].at[pl.ds(base * 16, n_w * 16)], wvs[li], sw))
        for h in stage:
            h.wait()

        first_k = min(_K, _DIMS[0][1] // _NW)
        pend = {(0, 0): issue_gather(0, 0, first_k),
                (0, 1): issue_gather(0, 1, first_k)}

        _coef_prep(wvs[0], tss[0], _DIMS[0][1] // _NW)

        fmt = plsc.PackFormat.INTERLEAVED
        owaits = {}
        for li, (din, dout) in enumerate(_DIMS):
            n_w = dout // _NW
            k = min(_K, n_w)
            n_chunks = n_w // k
            base = wid * n_w
            t0, t1, t2, t3 = tss[li]
            dst = dsts[li]
            for ck in range(n_chunks):
                p = ck % 2
                ha, hb = pend.pop((li, ck))
                ha.wait()
                hb.wait()
                if (li, ck - 2) in owaits:
                    owaits.pop((li, ck - 2)).wait()
                av, bv, ov = abufs[p], bbufs[p], obufs[p]

                def neuron_body(j, carry, _ck=ck, _k=k, _av=av, _bv=bv,
                                _ov=ov, _t0=t0, _t1=t1, _t2=t2, _t3=t3):
                    jj = _ck * _k + j
                    idx = jnp.full((_L,), jj, dtype=jnp.int32)
                    c0f = plsc.load_gather(_t0, [idx])
                    c1f = plsc.load_gather(_t1, [idx])
                    c2f = plsc.load_gather(_t2, [idx])
                    c3f = plsc.load_gather(_t3, [idx])
                    c0 = plsc.pack(c0f, c0f, format=fmt)
                    c1 = plsc.pack(c1f, c1f, format=fmt)
                    c2 = plsc.pack(c2f, c2f, format=fmt)
                    c3 = plsc.pack(c3f, c3f, format=fmt)
                    for v in range(_B2 // _L):
                        sl = pl.ds(v * _L, _L)
                        a = plsc.bitcast(_av[j, sl], jnp.bfloat16)
                        b = plsc.bitcast(_bv[j, sl], jnp.bfloat16)
                        r = (c0 + c1 * a) + (c2 + c3 * a) * b
                        _ov[j, sl] = plsc.bitcast(r, jnp.int32)
                    return carry

                lax.fori_loop(0, k, neuron_body, 0)
                owaits[(li, ck)] = pltpu.async_copy(
                    ov.at[pl.ds(0, k)], dst.at[pl.ds(base + ck * k, k)],
                    osems[p])
                if ck + 2 < n_chunks:
                    pend[(li, ck + 2)] = issue_gather(li, ck + 2, k)
            for key in list(owaits):
                owaits.pop(key).wait()
            if li + 1 < len(_DIMS):
                _coef_prep(wvs[li + 1], tss[li + 1], _DIMS[li + 1][1] // _NW)
                plsc.subcore_barrier()

                @pl.when(s == 0)
                def _(_li=li):
                    flagv[...] = jnp.full((_L,), _li + 1, jnp.int32)
                    pltpu.sync_copy(
                        flagv, flags.at[pl.ds((_li * _NC + c) * _L, _L)])
                    other = pl.ds((_li * _NC + (1 - c)) * _L, _L)

                    def poll(val):
                        pltpu.sync_copy(flags.at[other], pollv)
                        return jnp.max(pollv[...], axis=0)

                    lax.while_loop(lambda v: v <= _li, poll,
                                   jnp.zeros((), jnp.int32))

                plsc.subcore_barrier()
                n_w2 = _DIMS[li + 1][1] // _NW
                k2 = min(_K, n_w2)
                pend[(li + 1, 0)] = issue_gather(li + 1, 0, k2)
                if n_w2 // k2 > 1:
                    pend[(li + 1, 1)] = issue_gather(li + 1, 1, k2)

    return fused


_FUSED = _build()


def kernel(x, w0, a0, b0, w1, a1, b1, w2, a2, b2, w3, a3, b3):
    xb = x.T.astype(jnp.bfloat16).reshape(_DIMS[0][0], _B2, 2)
    xt = lax.bitcast_convert_type(xb, jnp.int32)
    ho, _ = _FUSED(
        xt, w0.reshape(-1), a0, b0, w1.reshape(-1), a1, b1,
        w2.reshape(-1), a2, b2, w3.reshape(-1), a3, b3)
    out = lax.bitcast_convert_type(ho, jnp.bfloat16).reshape(_DIMS[-1][1], _B)
    return out.T.astype(jnp.float32)
